# Initial kernel scaffold; baseline (speedup 1.0000x reference)
import jax
import jax.numpy as jnp
from jax.experimental import pallas as pl


def kernel(x):
    raise NotImplementedError("write your pallas kernel here")


# speedup vs baseline: 19.4600x; 19.4600x over previous
import functools

import jax
import jax.numpy as jnp
from jax import lax
from jax.experimental import pallas as pl
from jax.experimental.pallas import tpu as pltpu
from jax.experimental.pallas import tpu_sc as plsc

NC = 2
NS = 16
NW = NC * NS
ROWS = 65536
COLS = 100
MCOLS = 10
ROWS_PER = ROWS // NW
A_PER = ROWS_PER * MCOLS
CH = 512
NCH = ROWS_PER // CH


def _sc_partials(xflat):
    mesh = plsc.VectorSubcoreMesh(core_axis_name="c", subcore_axis_name="s")

    @functools.partial(
        pl.kernel,
        out_type=jax.ShapeDtypeStruct((NW, 16), jnp.float32),
        mesh=mesh,
        scratch_types=[
            pltpu.VMEM((CH * MCOLS + 16,), jnp.float32),
            pltpu.VMEM((CH * COLS + 16,), jnp.float32),
            pltpu.VMEM((16,), jnp.float32),
        ],
    )
    def k(xflat_hbm, out_hbm, a_v, b_v, res_v):
        wid = lax.axis_index("s") * NC + lax.axis_index("c")
        base_row = wid * ROWS_PER
        lane = lax.iota(jnp.int32, 16)
        mask = lane < MCOLS

        def chunk(c, acc):
            row0 = base_row + c * CH
            pltpu.sync_copy(xflat_hbm.at[pl.ds(row0 * COLS, CH * COLS)],
                            b_v.at[pl.ds(0, CH * COLS)])
            pltpu.sync_copy(xflat_hbm.at[pl.ds(row0 * MCOLS, CH * MCOLS)],
                            a_v.at[pl.ds(0, CH * MCOLS)])

            def body(q, acc2):
                va = a_v[pl.ds(q * MCOLS, 16)]
                vb = b_v[pl.ds(q * COLS, 16)]
                d = jnp.abs(va - vb)
                return acc2 + jnp.where(mask, d, 0.0)

            return lax.fori_loop(0, CH, body, acc)

        acc = lax.fori_loop(0, NCH, chunk, jnp.zeros((16,), jnp.float32))
        res_v[...] = acc
        pltpu.sync_copy(res_v, out_hbm.at[wid])

    return k(xflat)


def kernel(x):
    partials = _sc_partials(x.reshape(-1))
    return jnp.sum(partials)
